# baseline (speedup 1.0000x reference)
import functools

import jax
import jax.numpy as jnp
from jax import lax
from jax.experimental import pallas as pl
from jax.experimental.pallas import tpu as pltpu
from jax.experimental.pallas import tpu_sc as plsc

N_NODES = 10000
N_EDGES = 320000
D_FEAT = 128

NC, NS, L = 2, 16, 16
NW = NC * NS
E_PER_W = N_EDGES // NW
STEPS = E_PER_W // L
N_VECS = N_NODES // L
E_BUF = 10240


@functools.cache
def _scatter_w_kernel():
    mesh = plsc.VectorSubcoreMesh(core_axis_name="c", subcore_axis_name="s")

    @functools.partial(
        pl.kernel,
        mesh=mesh,
        compiler_params=pltpu.CompilerParams(needs_layout_passes=False),
        out_type=jax.ShapeDtypeStruct((NW, N_NODES), jnp.float32),
        scratch_types=[
            pltpu.VMEM((N_NODES,), jnp.float32),
            pltpu.VMEM((2, E_BUF), jnp.int32),
            pltpu.VMEM((E_PER_W,), jnp.float32),
        ],
    )
    def _scatter_w(edge_hbm, attr_hbm, out_hbm, acc_v, src_v, attr_v):
        wid = lax.axis_index("s") * NC + lax.axis_index("c")
        base = wid * E_PER_W
        base_al = pl.multiple_of(
            jnp.minimum((base // 128) * 128, N_EDGES - E_BUF), 128)
        off = base - base_al
        pltpu.sync_copy(edge_hbm.at[:, pl.ds(base_al, E_BUF)], src_v)
        pltpu.sync_copy(attr_hbm.at[pl.ds(base, E_PER_W)], attr_v)

        zeros = jnp.zeros((L,), jnp.float32)

        def zero_body(i, _):
            acc_v[pl.ds(i * L, L)] = zeros
            return 0

        lax.fori_loop(0, N_VECS, zero_body, 0)

        def body(i, _):
            idx = src_v[0, pl.ds(off + i * L, L)]
            a = attr_v[pl.ds(i * L, L)]
            plsc.addupdate_scatter(acc_v, [idx], a)
            return 0

        lax.fori_loop(0, STEPS, body, 0)
        pltpu.sync_copy(acc_v, out_hbm.at[wid])

    return _scatter_w


def _matvec_body(x_ref, w_ref, y_ref):
    y_ref[...] = jnp.sum(x_ref[...] * w_ref[...], axis=1, keepdims=True)


def _combine_body(p_ref, y_ref, b_ref, o_ref):
    w_nodes = jnp.sum(p_ref[...], axis=0, keepdims=True)
    s = jax.lax.dot_general(
        w_nodes, y_ref[...], (((1,), (0,)), ((), ())),
        precision=jax.lax.Precision.HIGHEST,
        preferred_element_type=jnp.float32)
    o_ref[...] = s * (1.0 / N_NODES) + b_ref[...]


def kernel(x, edge_index, edge_attr, W, b):
    partials = _scatter_w_kernel()(edge_index.astype(jnp.int32), edge_attr)
    y2d = pl.pallas_call(
        _matvec_body,
        out_shape=jax.ShapeDtypeStruct((N_NODES, 1), jnp.float32),
    )(x, W)
    out = pl.pallas_call(
        _combine_body,
        out_shape=jax.ShapeDtypeStruct((1, 1), jnp.float32),
    )(partials, y2d, b.reshape(1, 1))
    return out


# speedup vs baseline: 1.1018x; 1.1018x over previous
import functools

import jax
import jax.numpy as jnp
from jax import lax
from jax.experimental import pallas as pl
from jax.experimental.pallas import tpu as pltpu
from jax.experimental.pallas import tpu_sc as plsc

N_NODES = 10000
N_EDGES = 320000
D_FEAT = 128

NC, NS, L = 2, 16, 16
NW = NC * NS
E_PER_W = N_EDGES // NW
STEPS = E_PER_W // L
N_VECS = N_NODES // L
E_BUF = 10240


@functools.cache
def _scatter_w_kernel():
    mesh = plsc.VectorSubcoreMesh(core_axis_name="c", subcore_axis_name="s")

    @functools.partial(
        pl.kernel,
        mesh=mesh,
        compiler_params=pltpu.CompilerParams(needs_layout_passes=False),
        out_type=jax.ShapeDtypeStruct((NW, N_NODES), jnp.float32),
        scratch_types=[
            pltpu.VMEM((N_NODES,), jnp.float32),
            pltpu.VMEM((2, E_BUF), jnp.int32),
            pltpu.VMEM((E_PER_W,), jnp.float32),
        ],
    )
    def _scatter_w(edge_hbm, attr_hbm, out_hbm, acc_v, src_v, attr_v):
        wid = lax.axis_index("s") * NC + lax.axis_index("c")
        base = wid * E_PER_W
        base_al = pl.multiple_of(
            jnp.minimum((base // 128) * 128, N_EDGES - E_BUF), 128)
        off = base - base_al
        pltpu.sync_copy(edge_hbm.at[:, pl.ds(base_al, E_BUF)], src_v)
        pltpu.sync_copy(attr_hbm.at[pl.ds(base, E_PER_W)], attr_v)

        zeros = jnp.zeros((L,), jnp.float32)
        ZU = 25

        def zero_body(i, _):
            for j in range(ZU):
                acc_v[pl.ds((i * ZU + j) * L, L)] = zeros
            return 0

        lax.fori_loop(0, N_VECS // ZU, zero_body, 0)

        U = 25

        def body(i, _):
            for j in range(U):
                e = (i * U + j) * L
                idx = src_v[0, pl.ds(off + e, L)]
                a = attr_v[pl.ds(e, L)]
                plsc.addupdate_scatter(acc_v, [idx], a)
            return 0

        lax.fori_loop(0, STEPS // U, body, 0)
        pltpu.sync_copy(acc_v, out_hbm.at[wid])

    return _scatter_w


def _dense_body(p_ref, x_ref, w_ref, b_ref, o_ref):
    w_nodes = jnp.sum(p_ref[...], axis=0, keepdims=True)
    v = jax.lax.dot_general(
        w_nodes, x_ref[...], (((1,), (0,)), ((), ())),
        precision=jax.lax.Precision.HIGHEST,
        preferred_element_type=jnp.float32)
    o_ref[...] = jnp.sum(v * w_ref[...]) * (1.0 / N_NODES) + b_ref[...]


def kernel(x, edge_index, edge_attr, W, b):
    partials = _scatter_w_kernel()(edge_index.astype(jnp.int32), edge_attr)
    out = pl.pallas_call(
        _dense_body,
        out_shape=jax.ShapeDtypeStruct((1, 1), jnp.float32),
    )(partials, x, W, b.reshape(1, 1))
    return out


# speedup vs baseline: 1.2213x; 1.1084x over previous
import functools

import jax
import jax.numpy as jnp
from jax import lax
from jax.experimental import pallas as pl
from jax.experimental.pallas import tpu as pltpu
from jax.experimental.pallas import tpu_sc as plsc

N_NODES = 10000
N_EDGES = 320000
D_FEAT = 128

NC, NS, L = 2, 16, 16
NW = NC * NS
E_PER_W = N_EDGES // NW
STEPS = E_PER_W // L
N_VECS = N_NODES // L
E_BUF = 10240


@functools.cache
def _scatter_w_kernel():
    mesh = plsc.VectorSubcoreMesh(core_axis_name="c", subcore_axis_name="s")

    @functools.partial(
        pl.kernel,
        mesh=mesh,
        compiler_params=pltpu.CompilerParams(needs_layout_passes=False),
        out_type=jax.ShapeDtypeStruct((NW, N_NODES), jnp.float32),
        scratch_types=[
            pltpu.VMEM((N_NODES,), jnp.float32),
            pltpu.VMEM((2, E_BUF), jnp.int32),
            pltpu.VMEM((E_PER_W,), jnp.float32),
        ],
    )
    def _scatter_w(edge_hbm, attr_hbm, out_hbm, acc_v, src_v, attr_v):
        wid = lax.axis_index("s") * NC + lax.axis_index("c")
        base = wid * E_PER_W
        base_al = pl.multiple_of(
            jnp.minimum((base // 128) * 128, N_EDGES - E_BUF), 128)
        off = base - base_al
        pltpu.sync_copy(edge_hbm.at[:, pl.ds(base_al, E_BUF)], src_v)
        pltpu.sync_copy(attr_hbm.at[pl.ds(base, E_PER_W)], attr_v)

        zeros = jnp.zeros((L,), jnp.float32)
        ZU = 25

        def zero_body(i, _):
            for j in range(ZU):
                acc_v[pl.ds((i * ZU + j) * L, L)] = zeros
            return 0

        lax.fori_loop(0, N_VECS // ZU, zero_body, 0)

        U = 25

        def body(i, _):
            for j in range(U):
                e = (i * U + j) * L
                idx = src_v[0, pl.ds(off + e, L)]
                a = attr_v[pl.ds(e, L)]
                plsc.addupdate_scatter(acc_v, [idx], a)
            return 0

        lax.fori_loop(0, STEPS // U, body, 0)
        pltpu.sync_copy(acc_v, out_hbm.at[wid])

    return _scatter_w


def _matvec_t_body(w_ref, x_ref, y_ref):
    y_ref[...] = jax.lax.dot_general(
        w_ref[...], x_ref[...], (((1,), (1,)), ((), ())),
        precision=jax.lax.Precision.HIGHEST,
        preferred_element_type=jnp.float32)


def _combine_body(p_ref, y_ref, b_ref, o_ref):
    w_nodes = jnp.sum(p_ref[...], axis=0, keepdims=True)
    o_ref[...] = (
        jnp.sum(w_nodes * y_ref[...]) * (1.0 / N_NODES) + b_ref[...]
    )


def kernel(x, edge_index, edge_attr, W, b):
    partials = _scatter_w_kernel()(edge_index.astype(jnp.int32), edge_attr)
    y_t = pl.pallas_call(
        _matvec_t_body,
        out_shape=jax.ShapeDtypeStruct((1, N_NODES), jnp.float32),
    )(W, x)
    out = pl.pallas_call(
        _combine_body,
        out_shape=jax.ShapeDtypeStruct((1, 1), jnp.float32),
    )(partials, y_t, b.reshape(1, 1))
    return out


# speedup vs baseline: 1.2640x; 1.0349x over previous
import functools

import jax
import jax.numpy as jnp
from jax import lax
from jax.experimental import pallas as pl
from jax.experimental.pallas import tpu as pltpu
from jax.experimental.pallas import tpu_sc as plsc

N_NODES = 10000
N_EDGES = 320000
D_FEAT = 128

NC, NS, L = 2, 16, 16
NW = NC * NS
E_PER_W = N_EDGES // NW
STEPS = E_PER_W // L
N_VECS = N_NODES // L
E_BUF = 10240


@functools.cache
def _scatter_w_kernel():
    mesh = plsc.VectorSubcoreMesh(core_axis_name="c", subcore_axis_name="s")

    @functools.partial(
        pl.kernel,
        mesh=mesh,
        compiler_params=pltpu.CompilerParams(needs_layout_passes=False),
        out_type=jax.ShapeDtypeStruct((NW, N_NODES), jnp.float32),
        scratch_types=[
            pltpu.VMEM((N_NODES,), jnp.float32),
            pltpu.VMEM((2, E_BUF), jnp.int32),
            pltpu.VMEM((E_PER_W,), jnp.float32),
            pltpu.SemaphoreType.DMA,
            pltpu.SemaphoreType.DMA,
        ],
    )
    def _scatter_w(edge_hbm, attr_hbm, out_hbm, acc_v, src_v, attr_v, sem_e, sem_a):
        wid = lax.axis_index("s") * NC + lax.axis_index("c")
        base = wid * E_PER_W
        base_al = pl.multiple_of(
            jnp.minimum((base // 128) * 128, N_EDGES - E_BUF), 128)
        off = base - base_al
        cp_e = pltpu.async_copy(edge_hbm.at[:, pl.ds(base_al, E_BUF)], src_v, sem_e)
        cp_a = pltpu.async_copy(attr_hbm.at[pl.ds(base, E_PER_W)], attr_v, sem_a)

        zeros = jnp.zeros((L,), jnp.float32)
        ZU = 25

        def zero_body(i, _):
            for j in range(ZU):
                acc_v[pl.ds((i * ZU + j) * L, L)] = zeros
            return 0

        lax.fori_loop(0, N_VECS // ZU, zero_body, 0)
        cp_e.wait()
        cp_a.wait()

        U = 25

        def body(i, _):
            for j in range(U):
                e = (i * U + j) * L
                idx = src_v[0, pl.ds(off + e, L)]
                a = attr_v[pl.ds(e, L)]
                plsc.addupdate_scatter(acc_v, [idx], a)
            return 0

        lax.fori_loop(0, STEPS // U, body, 0)
        pltpu.sync_copy(acc_v, out_hbm.at[wid])

    return _scatter_w


def _matvec_t_body(w_ref, x_ref, y_ref):
    y_ref[...] = jax.lax.dot_general(
        w_ref[...], x_ref[...], (((1,), (1,)), ((), ())),
        precision=jax.lax.Precision.HIGHEST,
        preferred_element_type=jnp.float32)


def _combine_body(p_ref, y_ref, b_ref, o_ref):
    w_nodes = jnp.sum(p_ref[...], axis=0, keepdims=True)
    o_ref[...] = (
        jnp.sum(w_nodes * y_ref[...]) * (1.0 / N_NODES) + b_ref[...]
    )


def kernel(x, edge_index, edge_attr, W, b):
    partials = _scatter_w_kernel()(edge_index.astype(jnp.int32), edge_attr)
    y_t = pl.pallas_call(
        _matvec_t_body,
        out_shape=jax.ShapeDtypeStruct((1, N_NODES), jnp.float32),
    )(W, x)
    out = pl.pallas_call(
        _combine_body,
        out_shape=jax.ShapeDtypeStruct((1, 1), jnp.float32),
    )(partials, y_t, b.reshape(1, 1))
    return out


# speedup vs baseline: 1.3462x; 1.0651x over previous
import functools

import jax
import jax.numpy as jnp
from jax import lax
from jax.experimental import pallas as pl
from jax.experimental.pallas import tpu as pltpu
from jax.experimental.pallas import tpu_sc as plsc

N_NODES = 10000
N_EDGES = 320000
D_FEAT = 128

NC, NS, L = 2, 16, 16
NW = NC * NS
E_PER_W = N_EDGES // NW
STEPS = E_PER_W // L
N_VECS = N_NODES // L
E_BUF = 10240


@functools.cache
def _scatter_w_kernel():
    mesh = plsc.VectorSubcoreMesh(core_axis_name="c", subcore_axis_name="s")

    @functools.partial(
        pl.kernel,
        mesh=mesh,
        compiler_params=pltpu.CompilerParams(needs_layout_passes=False),
        out_type=jax.ShapeDtypeStruct((NW, N_NODES), jnp.float32),
        scratch_types=[
            pltpu.VMEM((N_NODES,), jnp.float32),
            pltpu.VMEM((2, E_BUF), jnp.int32),
            pltpu.VMEM((E_PER_W,), jnp.float32),
            pltpu.SemaphoreType.DMA,
            pltpu.SemaphoreType.DMA,
        ],
    )
    def _scatter_w(edge_hbm, attr_hbm, out_hbm, acc_v, src_v, attr_v, sem_e, sem_a):
        wid = lax.axis_index("s") * NC + lax.axis_index("c")
        base = wid * E_PER_W
        base_al = pl.multiple_of(
            jnp.minimum((base // 128) * 128, N_EDGES - E_BUF), 128)
        off = base - base_al
        cp_e = pltpu.async_copy(edge_hbm.at[:, pl.ds(base_al, E_BUF)], src_v, sem_e)
        cp_a = pltpu.async_copy(attr_hbm.at[pl.ds(base, E_PER_W)], attr_v, sem_a)

        zeros = jnp.zeros((L,), jnp.float32)

        @plsc.parallel_loop(0, N_VECS, unroll=25)
        def _zero(i):
            acc_v[pl.ds(i * L, L)] = zeros

        cp_e.wait()
        cp_a.wait()

        @plsc.parallel_loop(0, STEPS, unroll=25)
        def _scat(i):
            e = i * L
            idx = src_v[0, pl.ds(off + e, L)]
            a = attr_v[pl.ds(e, L)]
            plsc.addupdate_scatter(acc_v, [idx], a)

        pltpu.sync_copy(acc_v, out_hbm.at[wid])

    return _scatter_w


def _matvec_t_body(w_ref, x_ref, y_ref):
    y_ref[...] = jax.lax.dot_general(
        w_ref[...], x_ref[...], (((1,), (1,)), ((), ())),
        precision=jax.lax.Precision.HIGHEST,
        preferred_element_type=jnp.float32)


def _combine_body(p_ref, y_ref, b_ref, o_ref):
    w_nodes = jnp.sum(p_ref[...], axis=0, keepdims=True)
    o_ref[...] = (
        jnp.sum(w_nodes * y_ref[...]) * (1.0 / N_NODES) + b_ref[...]
    )


def kernel(x, edge_index, edge_attr, W, b):
    partials = _scatter_w_kernel()(edge_index.astype(jnp.int32), edge_attr)
    y_t = pl.pallas_call(
        _matvec_t_body,
        out_shape=jax.ShapeDtypeStruct((1, N_NODES), jnp.float32),
    )(W, x)
    out = pl.pallas_call(
        _combine_body,
        out_shape=jax.ShapeDtypeStruct((1, 1), jnp.float32),
    )(partials, y_t, b.reshape(1, 1))
    return out
